# baseline (speedup 1.0000x reference)
import jax
import jax.numpy as jnp
from jax import lax
from jax.experimental import pallas as pl
from jax.experimental.pallas import tpu as pltpu
from jax.experimental.pallas import tpu_sc as plsc

_NC = 2
_NS = 16
_NW = _NC * _NS
_LANES = 16

_BATCH = 4096
_HIST = 200
_EMBED = 64
_HIDDEN = 128
_BPW = _BATCH // _NW
_NV = _EMBED // _LANES

_G = 2
_NG = _BPW // _G
_GIDX = _G * _HIST
_GCHUNKS = ((0, 128), (128, 128), (256, 128), (384, 16))


def _gather_group(xv, table_hbm, buf, sem, g):
    base = g * _GIDX
    for off, n in _GCHUNKS:
        pltpu.async_copy(
            table_hbm.at[xv.at[pl.ds(base + off, n)]],
            buf.at[pl.ds(off, n)],
            sem,
        )


def _wait_group(xv, table_hbm, buf, sem, g):
    base = g * _GIDX
    for off, n in _GCHUNKS:
        pltpu.make_async_copy(
            table_hbm.at[xv.at[pl.ds(base + off, n)]],
            buf.at[pl.ds(off, n)],
            sem,
        ).wait()


def _reduce_group(buf, outv, out_row):
    scale = jnp.float32(1.0 / _HIST)
    for r in range(_G):
        rbase = r * _HIST

        def body(li, acc, rbase=rbase):
            res = acc
            for u in range(4):
                row = rbase + li * 4 + u
                res = tuple(
                    res[i] + buf[row, pl.ds(_LANES * i, _LANES)]
                    for i in range(_NV)
                )
            return res

        acc = lax.fori_loop(
            0, _HIST // 4, body,
            tuple(jnp.zeros((_LANES,), jnp.float32) for _ in range(_NV)),
        )
        for i in range(_NV):
            outv[out_row + r, pl.ds(_LANES * i, _LANES)] = acc[i] * scale


def _pool_body(xflat_hbm, table_hbm, out_hbm, xv, buf0, buf1, outv,
               sem0, sem1):
    wid = lax.axis_index("s") * _NC + lax.axis_index("c")
    base = wid * _BPW
    pltpu.sync_copy(xflat_hbm.at[pl.ds(base * _HIST, _BPW * _HIST)], xv)

    _gather_group(xv, table_hbm, buf0, sem0, 0)

    def outer(k, carry):
        g_even = 2 * k
        g_odd = g_even + 1
        _gather_group(xv, table_hbm, buf1, sem1, g_odd)
        _wait_group(xv, table_hbm, buf0, sem0, g_even)
        _reduce_group(buf0, outv, g_even * _G)
        g_next = jnp.minimum(g_even + 2, _NG - 2)
        _gather_group(xv, table_hbm, buf0, sem0, g_next)
        _wait_group(xv, table_hbm, buf1, sem1, g_odd)
        _reduce_group(buf1, outv, g_odd * _G)
        return carry

    lax.fori_loop(0, _NG // 2, outer, 0)
    _wait_group(xv, table_hbm, buf0, sem0, _NG - 2)

    pltpu.sync_copy(outv, out_hbm.at[pl.ds(base, _BPW)])


def _sc_pool(xflat, table):
    mesh = plsc.VectorSubcoreMesh(core_axis_name="c", subcore_axis_name="s")
    f = pl.kernel(
        _pool_body,
        out_type=jax.ShapeDtypeStruct((_BATCH, _EMBED), jnp.float32),
        mesh=mesh,
        scratch_types=[
            pltpu.VMEM((_BPW * _HIST,), jnp.int32),
            pltpu.VMEM((_GIDX, _EMBED), jnp.float32),
            pltpu.VMEM((_GIDX, _EMBED), jnp.float32),
            pltpu.VMEM((_BPW, _EMBED), jnp.float32),
            pltpu.SemaphoreType.DMA,
            pltpu.SemaphoreType.DMA,
        ],
        compiler_params=pltpu.CompilerParams(use_tc_tiling_on_sc=False),
    )
    return f(xflat, table)


def _tr_body(tT_ref, o_ref):
    o_ref[...] = tT_ref[...].T


def _tc_transpose(tableT):
    vocab = tableT.shape[1]
    blk = 2048
    grid = (vocab + blk - 1) // blk
    return pl.pallas_call(
        _tr_body,
        grid=(grid,),
        in_specs=[pl.BlockSpec((_EMBED, blk), lambda i: (0, i))],
        out_specs=pl.BlockSpec((blk, _EMBED), lambda i: (i, 0)),
        out_shape=jax.ShapeDtypeStruct((vocab, _EMBED), jnp.float32),
    )(tableT)


def _mm_body(p_ref, w_ref, b_ref, o_ref):
    o_ref[...] = (
        jnp.dot(p_ref[...], w_ref[...], preferred_element_type=jnp.float32)
        + b_ref[...]
    )


def _tc_matmul(pooled, W, b):
    blk = 512
    return pl.pallas_call(
        _mm_body,
        grid=(_BATCH // blk,),
        in_specs=[
            pl.BlockSpec((blk, _EMBED), lambda i: (i, 0)),
            pl.BlockSpec((_EMBED, _HIDDEN), lambda i: (0, 0)),
            pl.BlockSpec((1, _HIDDEN), lambda i: (0, 0)),
        ],
        out_specs=pl.BlockSpec((blk, _HIDDEN), lambda i: (i, 0)),
        out_shape=jax.ShapeDtypeStruct((_BATCH, _HIDDEN), jnp.float32),
    )(pooled, W, b.reshape(1, _HIDDEN))


def kernel(x, table, W, b):
    xflat = x.astype(jnp.int32).reshape(-1)
    table_rm = _tc_transpose(table.T)
    pooled = _sc_pool(xflat, table_rm)
    return _tc_matmul(pooled, W, b)


# speedup vs baseline: 1.2568x; 1.2568x over previous
import jax
import jax.numpy as jnp
from jax import lax
from jax.experimental import pallas as pl
from jax.experimental.pallas import tpu as pltpu
from jax.experimental.pallas import tpu_sc as plsc

_NC = 2
_NS = 16
_NW = _NC * _NS
_LANES = 16

_BATCH = 4096
_HIST = 200
_EMBED = 64
_HIDDEN = 128
_BPW = _BATCH // _NW
_NV = _EMBED // _LANES

_G = 2
_NG = _BPW // _G
_GIDX = _G * _HIST
_GCHUNKS = ((0, 128), (128, 128), (256, 128), (384, 16))


def _gather_group(xv, table_hbm, buf, sem, g):
    base = g * _GIDX
    for off, n in _GCHUNKS:
        pltpu.async_copy(
            table_hbm.at[xv.at[pl.ds(base + off, n)]],
            buf.at[pl.ds(off, n)],
            sem,
        )


def _wait_group(xv, table_hbm, buf, sem, g):
    base = g * _GIDX
    for off, n in _GCHUNKS:
        pltpu.make_async_copy(
            table_hbm.at[xv.at[pl.ds(base + off, n)]],
            buf.at[pl.ds(off, n)],
            sem,
        ).wait()


def _reduce_group(buf, outv, out_row):
    scale = jnp.float32(1.0 / _HIST)
    for r in range(_G):
        rbase = r * _HIST

        def body(li, acc, rbase=rbase):
            res = acc
            for u in range(4):
                row = rbase + li * 4 + u
                res = tuple(
                    res[i] + buf[row, pl.ds(_LANES * i, _LANES)]
                    for i in range(_NV)
                )
            return res

        acc = lax.fori_loop(
            0, _HIST // 4, body,
            tuple(jnp.zeros((_LANES,), jnp.float32) for _ in range(_NV)),
        )
        for i in range(_NV):
            outv[out_row + r, pl.ds(_LANES * i, _LANES)] = acc[i] * scale


def _pool_body(xflat_hbm, table_hbm, out_hbm, xv, buf0, buf1, outv,
               sem0, sem1):
    wid = lax.axis_index("s") * _NC + lax.axis_index("c")
    base = wid * _BPW
    pltpu.sync_copy(xflat_hbm.at[pl.ds(base * _HIST, _BPW * _HIST)], xv)

    _gather_group(xv, table_hbm, buf0, sem0, 0)

    def outer(k, carry):
        g_even = 2 * k
        g_odd = g_even + 1
        _gather_group(xv, table_hbm, buf1, sem1, g_odd)
        _wait_group(xv, table_hbm, buf0, sem0, g_even)
        _reduce_group(buf0, outv, g_even * _G)
        g_next = jnp.minimum(g_even + 2, _NG - 2)
        _gather_group(xv, table_hbm, buf0, sem0, g_next)
        _wait_group(xv, table_hbm, buf1, sem1, g_odd)
        _reduce_group(buf1, outv, g_odd * _G)
        return carry

    lax.fori_loop(0, _NG // 2, outer, 0)
    _wait_group(xv, table_hbm, buf0, sem0, _NG - 2)

    pltpu.sync_copy(outv, out_hbm.at[pl.ds(base, _BPW)])


def _sc_pool(xflat, table):
    mesh = plsc.VectorSubcoreMesh(core_axis_name="c", subcore_axis_name="s")
    f = pl.kernel(
        _pool_body,
        out_type=jax.ShapeDtypeStruct((_BATCH, _EMBED), jnp.float32),
        mesh=mesh,
        scratch_types=[
            pltpu.VMEM((_BPW * _HIST,), jnp.int32),
            pltpu.VMEM((_GIDX, _EMBED), jnp.float32),
            pltpu.VMEM((_GIDX, _EMBED), jnp.float32),
            pltpu.VMEM((_BPW, _EMBED), jnp.float32),
            pltpu.SemaphoreType.DMA,
            pltpu.SemaphoreType.DMA,
        ],
        compiler_params=pltpu.CompilerParams(use_tc_tiling_on_sc=False),
    )
    return f(xflat, table)


def _tr_body(tT_ref, o_ref):
    o_ref[...] = tT_ref[...].T


def _tc_transpose(tableT):
    vocab = tableT.shape[1]
    blk = 8192
    grid = (vocab + blk - 1) // blk
    return pl.pallas_call(
        _tr_body,
        grid=(grid,),
        in_specs=[pl.BlockSpec((_EMBED, blk), lambda i: (0, i))],
        out_specs=pl.BlockSpec((blk, _EMBED), lambda i: (i, 0)),
        out_shape=jax.ShapeDtypeStruct((vocab, _EMBED), jnp.float32),
    )(tableT)


def _mm_body(p_ref, w_ref, b_ref, o_ref):
    o_ref[...] = (
        jnp.dot(p_ref[...], w_ref[...], preferred_element_type=jnp.float32)
        + b_ref[...]
    )


def _tc_matmul(pooled, W, b):
    blk = 512
    return pl.pallas_call(
        _mm_body,
        grid=(_BATCH // blk,),
        in_specs=[
            pl.BlockSpec((blk, _EMBED), lambda i: (i, 0)),
            pl.BlockSpec((_EMBED, _HIDDEN), lambda i: (0, 0)),
            pl.BlockSpec((1, _HIDDEN), lambda i: (0, 0)),
        ],
        out_specs=pl.BlockSpec((blk, _HIDDEN), lambda i: (i, 0)),
        out_shape=jax.ShapeDtypeStruct((_BATCH, _HIDDEN), jnp.float32),
    )(pooled, W, b.reshape(1, _HIDDEN))


def kernel(x, table, W, b):
    xflat = x.astype(jnp.int32).reshape(-1)
    table_rm = _tc_transpose(table.T)
    pooled = _sc_pool(xflat, table_rm)
    return _tc_matmul(pooled, W, b)


# speedup vs baseline: 1.3644x; 1.0856x over previous
import jax
import jax.numpy as jnp
from jax import lax
from jax.experimental import pallas as pl
from jax.experimental.pallas import tpu as pltpu
from jax.experimental.pallas import tpu_sc as plsc

_NC = 2
_NS = 16
_NW = _NC * _NS
_LANES = 16

_BATCH = 4096
_HIST = 200
_EMBED = 64
_HIDDEN = 128
_BPW = _BATCH // _NW
_NV = _EMBED // _LANES

_G = 2
_NG = _BPW // _G
_GIDX = _G * _HIST
_GCHUNKS = ((0, 128), (128, 128), (256, 128), (384, 16))


def _gather_group(xv, table_hbm, buf, sem, g):
    base = g * _GIDX
    for off, n in _GCHUNKS:
        pltpu.async_copy(
            table_hbm.at[xv.at[pl.ds(base + off, n)]],
            buf.at[pl.ds(off, n)],
            sem,
        )


def _wait_group(xv, table_hbm, buf, sem, g):
    base = g * _GIDX
    for off, n in _GCHUNKS:
        pltpu.make_async_copy(
            table_hbm.at[xv.at[pl.ds(base + off, n)]],
            buf.at[pl.ds(off, n)],
            sem,
        ).wait()


def _reduce_group(buf, outv, out_row):
    scale = jnp.float32(1.0 / _HIST)
    for r in range(_G):
        rbase = r * _HIST

        def body(li, acc, rbase=rbase):
            res = acc
            for u in range(4):
                row = rbase + li * 4 + u
                res = tuple(
                    res[i] + buf[row, pl.ds(_LANES * i, _LANES)]
                    for i in range(_NV)
                )
            return res

        acc = lax.fori_loop(
            0, _HIST // 4, body,
            tuple(jnp.zeros((_LANES,), jnp.float32) for _ in range(_NV)),
        )
        for i in range(_NV):
            outv[out_row + r, pl.ds(_LANES * i, _LANES)] = acc[i] * scale


def _pool_body(xflat_hbm, table_hbm, out_hbm, xv, buf0, buf1, outv,
               sem0, sem1):
    wid = lax.axis_index("s") * _NC + lax.axis_index("c")
    base = wid * _BPW
    pltpu.sync_copy(xflat_hbm.at[pl.ds(base * _HIST, _BPW * _HIST)], xv)

    _gather_group(xv, table_hbm, buf0, sem0, 0)

    def outer(k, carry):
        g_even = 2 * k
        g_odd = g_even + 1
        _gather_group(xv, table_hbm, buf1, sem1, g_odd)
        _wait_group(xv, table_hbm, buf0, sem0, g_even)
        _reduce_group(buf0, outv, g_even * _G)
        g_next = jnp.minimum(g_even + 2, _NG - 2)
        _gather_group(xv, table_hbm, buf0, sem0, g_next)
        _wait_group(xv, table_hbm, buf1, sem1, g_odd)
        _reduce_group(buf1, outv, g_odd * _G)
        return carry

    lax.fori_loop(0, _NG // 2, outer, 0)
    _wait_group(xv, table_hbm, buf0, sem0, _NG - 2)

    pltpu.sync_copy(outv, out_hbm.at[pl.ds(base, _BPW)])


def _sc_pool(xflat, table):
    mesh = plsc.VectorSubcoreMesh(core_axis_name="c", subcore_axis_name="s")
    f = pl.kernel(
        _pool_body,
        out_type=jax.ShapeDtypeStruct((_BATCH, _EMBED), jnp.float32),
        mesh=mesh,
        scratch_types=[
            pltpu.VMEM((_BPW * _HIST,), jnp.int32),
            pltpu.VMEM((_GIDX, _EMBED), jnp.float32),
            pltpu.VMEM((_GIDX, _EMBED), jnp.float32),
            pltpu.VMEM((_BPW, _EMBED), jnp.float32),
            pltpu.SemaphoreType.DMA,
            pltpu.SemaphoreType.DMA,
        ],
        compiler_params=pltpu.CompilerParams(use_tc_tiling_on_sc=False),
    )
    return f(xflat, table)


def _tr_body(tT_ref, o_ref):
    o_ref[...] = tT_ref[...].T


def _tc_transpose(tableT):
    vocab = tableT.shape[1]
    blk = 8192
    grid = (vocab + blk - 1) // blk
    return pl.pallas_call(
        _tr_body,
        grid=(grid,),
        in_specs=[pl.BlockSpec((_EMBED, blk), lambda i: (0, i))],
        out_specs=pl.BlockSpec((blk, _EMBED), lambda i: (i, 0)),
        out_shape=jax.ShapeDtypeStruct((vocab, _EMBED), jnp.float32),
    )(tableT)


def _mm_body(p_ref, w_ref, b_ref, o_ref):
    o_ref[...] = (
        jnp.dot(p_ref[...], w_ref[...], preferred_element_type=jnp.float32)
        + b_ref[...]
    )


def _tc_matmul(pooled, W, b):
    blk = 512
    return pl.pallas_call(
        _mm_body,
        grid=(_BATCH // blk,),
        in_specs=[
            pl.BlockSpec((blk, _EMBED), lambda i: (i, 0)),
            pl.BlockSpec((_EMBED, _HIDDEN), lambda i: (0, 0)),
            pl.BlockSpec((1, _HIDDEN), lambda i: (0, 0)),
        ],
        out_specs=pl.BlockSpec((blk, _HIDDEN), lambda i: (i, 0)),
        out_shape=jax.ShapeDtypeStruct((_BATCH, _HIDDEN), jnp.float32),
    )(pooled, W, b.reshape(1, _HIDDEN))


def kernel(x, table, W, b):
    xflat = x.astype(jnp.int32).reshape(-1)
    pooled = _sc_pool(xflat, table)
    return _tc_matmul(pooled, W, b)
